# baseline (speedup 1.0000x reference)
import jax
import jax.numpy as jnp
from jax import lax
from jax.experimental import pallas as pl
from jax.experimental.pallas import tpu as pltpu
from jax.experimental.pallas import tpu_sc as plsc

N_NODES = 10000
N_EDGES = 320000
D = 128

NC = 2
NS = 16
NW = NC * NS

CHUNK = 32
E_PER_T = N_EDGES // NW
E_PER_T_PAD = 10240
NCHUNK = E_PER_T_PAD // CHUNK
SUPER = 8
NSUPER = NCHUNK // SUPER
NBUF = 8
LOOK = 4
NBANK = 5
AGG_ROWS = 10240
ROWS_PER_TILE = AGG_ROWS // NS
DUMMY_DST = N_NODES


def _sc_agg_body(x_hbm, src_hbm, dst_hbm, zero_hbm, out_hbm,
                 srcb, dstb, rows, agg, sem_i, sem_g, sem_s):
    c = lax.axis_index("c")
    s = lax.axis_index("s")
    wid = s * NC + c
    base = wid * NCHUNK

    pltpu.sync_copy(zero_hbm, agg.at[pl.ds(s * ROWS_PER_TILE, ROWS_PER_TILE)])
    plsc.subcore_barrier()

    pltpu.sync_copy(src_hbm.at[pl.ds(base, SUPER)], srcb.at[0])
    pltpu.sync_copy(dst_hbm.at[pl.ds(base, SUPER)], dstb.at[0])
    for n in range(1, 4):
        pltpu.async_copy(src_hbm.at[pl.ds(base + n * SUPER, SUPER)],
                         srcb.at[n], sem_i)
        pltpu.async_copy(dst_hbm.at[pl.ds(base + n * SUPER, SUPER)],
                         dstb.at[n], sem_i)
    for g in range(LOOK):
        pltpu.async_copy(x_hbm.at[srcb.at[0, g]], rows.at[g], sem_g.at[g])

    def outer(k, carry):
        q = k % NBANK
        qn = (k + 1) % NBANK
        for j in range(SUPER):
            p = j % NBUF
            pf = (j + LOOK) % NBUF
            if j + LOOK < SUPER:
                if j + LOOK < NBUF:
                    @pl.when(k > 0)
                    def _():
                        pltpu.make_async_copy(rows.at[pf],
                                              agg.at[dstb.at[q, 0]],
                                              sem_s.at[pf]).wait()
                else:
                    pltpu.make_async_copy(rows.at[pf], agg.at[dstb.at[q, 0]],
                                          sem_s.at[pf]).wait()
                pltpu.async_copy(x_hbm.at[srcb.at[q, j + LOOK]],
                                 rows.at[pf], sem_g.at[pf])
            elif j + LOOK == SUPER:
                @pl.when(k + 1 < NSUPER)
                def _():
                    pltpu.make_async_copy(rows.at[pf], agg.at[dstb.at[q, 0]],
                                          sem_s.at[pf]).wait()
                    pltpu.make_async_copy(
                        src_hbm.at[pl.ds(base, SUPER)], srcb.at[qn],
                        sem_i).wait()
                    pltpu.make_async_copy(
                        dst_hbm.at[pl.ds(base, SUPER)], dstb.at[qn],
                        sem_i).wait()
                    pltpu.async_copy(x_hbm.at[srcb.at[qn, j + LOOK - SUPER]],
                                     rows.at[pf], sem_g.at[pf])
            else:
                @pl.when(k + 1 < NSUPER)
                def _():
                    pltpu.make_async_copy(rows.at[pf], agg.at[dstb.at[q, 0]],
                                          sem_s.at[pf]).wait()
                    pltpu.async_copy(x_hbm.at[srcb.at[qn, j + LOOK - SUPER]],
                                     rows.at[pf], sem_g.at[pf])
            pltpu.make_async_copy(x_hbm.at[srcb.at[0, 0]], rows.at[p],
                                  sem_g.at[p]).wait()
            pltpu.async_copy(rows.at[p], agg.at[dstb.at[q, j]], sem_s.at[p],
                             add=True)
        @pl.when(k + 4 < NSUPER)
        def _():
            pltpu.async_copy(src_hbm.at[pl.ds(base + (k + 4) * SUPER, SUPER)],
                             srcb.at[(k + 4) % NBANK], sem_i)
            pltpu.async_copy(dst_hbm.at[pl.ds(base + (k + 4) * SUPER, SUPER)],
                             dstb.at[(k + 4) % NBANK], sem_i)
        return carry

    lax.fori_loop(0, NSUPER, outer, 0)

    for b in range(NBUF):
        pltpu.make_async_copy(rows.at[b], agg.at[dstb.at[0, 0]],
                              sem_s.at[b]).wait()
    plsc.subcore_barrier()

    pltpu.sync_copy(agg.at[pl.ds(s * ROWS_PER_TILE, ROWS_PER_TILE)],
                    out_hbm.at[c, pl.ds(s * ROWS_PER_TILE, ROWS_PER_TILE)])


def _sc_aggregate(x, src2d, dst2d, zeros_stripe):
    mesh = plsc.VectorSubcoreMesh(core_axis_name="c", subcore_axis_name="s",
                                  num_cores=NC, num_subcores=NS)
    return pl.kernel(
        _sc_agg_body,
        out_type=jax.ShapeDtypeStruct((NC, AGG_ROWS, D), jnp.float32),
        mesh=mesh,
        scratch_types=[
            pltpu.VMEM((NBANK, SUPER, CHUNK), jnp.int32),
            pltpu.VMEM((NBANK, SUPER, CHUNK), jnp.int32),
            pltpu.VMEM((NBUF, CHUNK, D), jnp.float32),
            pltpu.VMEM_SHARED((AGG_ROWS, D), jnp.float32),
            pltpu.SemaphoreType.DMA,
            pltpu.SemaphoreType.DMA((NBUF,)),
            pltpu.SemaphoreType.DMA((NBUF,)),
        ],
    )(x, src2d, dst2d, zeros_stripe)


def _mlp_body(eps_ref, x_ref, agg_ref, w1_ref, b1_ref, w2_ref, b2_ref, o_ref):
    eps = eps_ref[0]
    h = (1.0 + eps) * x_ref[...] + agg_ref[0] + agg_ref[1]
    h1 = lax.dot_general(h, w1_ref[...], (((1,), (1,)), ((), ())),
                         preferred_element_type=jnp.float32) + b1_ref[...]
    h1 = jnp.maximum(h1, 0.0)
    o_ref[...] = lax.dot_general(h1, w2_ref[...], (((1,), (1,)), ((), ())),
                                 preferred_element_type=jnp.float32) + b2_ref[...]


def _mlp(x, agg2, W1, b1, W2, b2, eps):
    blk = 2000
    grid = (N_NODES // blk,)
    return pl.pallas_call(
        _mlp_body,
        grid=grid,
        in_specs=[
            pl.BlockSpec(memory_space=pltpu.SMEM),
            pl.BlockSpec((blk, D), lambda i: (i, 0)),
            pl.BlockSpec((NC, blk, D), lambda i: (0, i, 0)),
            pl.BlockSpec((D, D), lambda i: (0, 0)),
            pl.BlockSpec((1, D), lambda i: (0, 0)),
            pl.BlockSpec((D, D), lambda i: (0, 0)),
            pl.BlockSpec((1, D), lambda i: (0, 0)),
        ],
        out_specs=pl.BlockSpec((blk, D), lambda i: (i, 0)),
        out_shape=jax.ShapeDtypeStruct((N_NODES, D), jnp.float32),
    )(eps.reshape(1), x, agg2, W1, b1.reshape(1, D), W2, b2.reshape(1, D))


def kernel(x, edge_index, W1, b1, W2, b2, eps):
    ei = edge_index.astype(jnp.int32)
    pad = E_PER_T_PAD - E_PER_T
    src2d = jnp.concatenate(
        [ei[0].reshape(NW, E_PER_T),
         jnp.zeros((NW, pad), jnp.int32)], axis=1).reshape(NW * NCHUNK, CHUNK)
    dst2d = jnp.concatenate(
        [ei[1].reshape(NW, E_PER_T),
         jnp.full((NW, pad), DUMMY_DST, jnp.int32)],
        axis=1).reshape(NW * NCHUNK, CHUNK)
    zeros_stripe = jnp.zeros((ROWS_PER_TILE, D), jnp.float32)
    agg2 = _sc_aggregate(x, src2d, dst2d, zeros_stripe)
    return _mlp(x, agg2, W1, b1, W2, b2, eps)


# speedup vs baseline: 1.0420x; 1.0420x over previous
import jax
import jax.numpy as jnp
from jax import lax
from jax.experimental import pallas as pl
from jax.experimental.pallas import tpu as pltpu
from jax.experimental.pallas import tpu_sc as plsc

N_NODES = 10000
N_EDGES = 320000
D = 128

NC = 2
NS = 16
NW = NC * NS

CHUNK = 64
E_PER_T = N_EDGES // NW
NCHUNK = 160
E_PER_T_PAD = NCHUNK * CHUNK
SUPER = 8
NSUPER = NCHUNK // SUPER
NBUF = 4
LOOK = 3
NBANK = 5
AGG_ROWS = 10240
ROWS_PER_TILE = AGG_ROWS // NS
DUMMY_DST = N_NODES


def _sc_agg_body(x_hbm, src_hbm, dst_hbm, zero_hbm, out_hbm,
                 srcb, dstb, rows, agg, sem_i, sem_g, sem_s, sem_z):
    c = lax.axis_index("c")
    s = lax.axis_index("s")
    wid = s * NC + c
    base = wid * NCHUNK

    pltpu.sync_copy(zero_hbm, rows.at[0])
    for t in range(ROWS_PER_TILE // CHUNK):
        pltpu.async_copy(
            rows.at[0],
            agg.at[pl.ds(s * ROWS_PER_TILE + t * CHUNK, CHUNK)], sem_z)

    pltpu.sync_copy(src_hbm.at[pl.ds(base, SUPER)], srcb.at[0])
    pltpu.sync_copy(dst_hbm.at[pl.ds(base, SUPER)], dstb.at[0])
    for n in range(1, 4):
        pltpu.async_copy(src_hbm.at[pl.ds(base + n * SUPER, SUPER)],
                         srcb.at[n], sem_i)
        pltpu.async_copy(dst_hbm.at[pl.ds(base + n * SUPER, SUPER)],
                         dstb.at[n], sem_i)
    for g in range(1, LOOK):
        pltpu.async_copy(x_hbm.at[srcb.at[0, g]], rows.at[g], sem_g.at[g])
    for t in range(ROWS_PER_TILE // CHUNK):
        pltpu.make_async_copy(
            rows.at[0], agg.at[pl.ds(s * ROWS_PER_TILE, CHUNK)], sem_z).wait()
    pltpu.async_copy(x_hbm.at[srcb.at[0, 0]], rows.at[0], sem_g.at[0])
    plsc.subcore_barrier()

    def outer(k, carry):
        q = k % NBANK
        qn = (k + 1) % NBANK
        for j in range(SUPER):
            p = j % NBUF
            pf = (j + LOOK) % NBUF
            if j == 0:
                @pl.when(k > 0)
                def _():
                    pltpu.make_async_copy(rows.at[pf], agg.at[dstb.at[q, 0]],
                                          sem_s.at[pf]).wait()
                pltpu.async_copy(x_hbm.at[srcb.at[q, j + LOOK]],
                                 rows.at[pf], sem_g.at[pf])
            elif j < SUPER - LOOK:
                pltpu.make_async_copy(rows.at[pf], agg.at[dstb.at[q, 0]],
                                      sem_s.at[pf]).wait()
                pltpu.async_copy(x_hbm.at[srcb.at[q, j + LOOK]],
                                 rows.at[pf], sem_g.at[pf])
            elif j == SUPER - LOOK:
                @pl.when(k + 1 < NSUPER)
                def _():
                    pltpu.make_async_copy(rows.at[pf], agg.at[dstb.at[q, 0]],
                                          sem_s.at[pf]).wait()
                    pltpu.make_async_copy(
                        src_hbm.at[pl.ds(base, SUPER)], srcb.at[qn],
                        sem_i).wait()
                    pltpu.make_async_copy(
                        dst_hbm.at[pl.ds(base, SUPER)], dstb.at[qn],
                        sem_i).wait()
                    pltpu.async_copy(x_hbm.at[srcb.at[qn, j + LOOK - SUPER]],
                                     rows.at[pf], sem_g.at[pf])
            else:
                @pl.when(k + 1 < NSUPER)
                def _():
                    pltpu.make_async_copy(rows.at[pf], agg.at[dstb.at[q, 0]],
                                          sem_s.at[pf]).wait()
                    pltpu.async_copy(x_hbm.at[srcb.at[qn, j + LOOK - SUPER]],
                                     rows.at[pf], sem_g.at[pf])
            pltpu.make_async_copy(x_hbm.at[srcb.at[0, 0]], rows.at[p],
                                  sem_g.at[p]).wait()
            pltpu.async_copy(rows.at[p], agg.at[dstb.at[q, j]], sem_s.at[p],
                             add=True)
        @pl.when(k + 4 < NSUPER)
        def _():
            pltpu.async_copy(src_hbm.at[pl.ds(base + (k + 4) * SUPER, SUPER)],
                             srcb.at[(k + 4) % NBANK], sem_i)
            pltpu.async_copy(dst_hbm.at[pl.ds(base + (k + 4) * SUPER, SUPER)],
                             dstb.at[(k + 4) % NBANK], sem_i)
        return carry

    lax.fori_loop(0, NSUPER, outer, 0)

    for b in range(NBUF):
        pltpu.make_async_copy(rows.at[b], agg.at[dstb.at[0, 0]],
                              sem_s.at[b]).wait()
    plsc.subcore_barrier()

    pltpu.sync_copy(agg.at[pl.ds(s * ROWS_PER_TILE, ROWS_PER_TILE)],
                    out_hbm.at[c, pl.ds(s * ROWS_PER_TILE, ROWS_PER_TILE)])


def _sc_aggregate(x, src2d, dst2d, zeros_chunk):
    mesh = plsc.VectorSubcoreMesh(core_axis_name="c", subcore_axis_name="s",
                                  num_cores=NC, num_subcores=NS)
    return pl.kernel(
        _sc_agg_body,
        out_type=jax.ShapeDtypeStruct((NC, AGG_ROWS, D), jnp.float32),
        mesh=mesh,
        scratch_types=[
            pltpu.VMEM((NBANK, SUPER, CHUNK), jnp.int32),
            pltpu.VMEM((NBANK, SUPER, CHUNK), jnp.int32),
            pltpu.VMEM((NBUF, CHUNK, D), jnp.float32),
            pltpu.VMEM_SHARED((AGG_ROWS, D), jnp.float32),
            pltpu.SemaphoreType.DMA,
            pltpu.SemaphoreType.DMA((NBUF,)),
            pltpu.SemaphoreType.DMA((NBUF,)),
            pltpu.SemaphoreType.DMA,
        ],
    )(x, src2d, dst2d, zeros_chunk)


def _mlp_body(eps_ref, x_ref, agg_ref, w1_ref, b1_ref, w2_ref, b2_ref, o_ref):
    eps = eps_ref[0]
    h = (1.0 + eps) * x_ref[...] + agg_ref[0] + agg_ref[1]
    h1 = lax.dot_general(h, w1_ref[...], (((1,), (1,)), ((), ())),
                         preferred_element_type=jnp.float32) + b1_ref[...]
    h1 = jnp.maximum(h1, 0.0)
    o_ref[...] = lax.dot_general(h1, w2_ref[...], (((1,), (1,)), ((), ())),
                                 preferred_element_type=jnp.float32) + b2_ref[...]


def _mlp(x, agg2, W1, b1, W2, b2, eps):
    blk = 2000
    grid = (N_NODES // blk,)
    return pl.pallas_call(
        _mlp_body,
        grid=grid,
        in_specs=[
            pl.BlockSpec(memory_space=pltpu.SMEM),
            pl.BlockSpec((blk, D), lambda i: (i, 0)),
            pl.BlockSpec((NC, blk, D), lambda i: (0, i, 0)),
            pl.BlockSpec((D, D), lambda i: (0, 0)),
            pl.BlockSpec((1, D), lambda i: (0, 0)),
            pl.BlockSpec((D, D), lambda i: (0, 0)),
            pl.BlockSpec((1, D), lambda i: (0, 0)),
        ],
        out_specs=pl.BlockSpec((blk, D), lambda i: (i, 0)),
        out_shape=jax.ShapeDtypeStruct((N_NODES, D), jnp.float32),
    )(eps.reshape(1), x, agg2, W1, b1.reshape(1, D), W2, b2.reshape(1, D))


def kernel(x, edge_index, W1, b1, W2, b2, eps):
    ei = edge_index.astype(jnp.int32)
    pad = E_PER_T_PAD - E_PER_T
    src2d = jnp.concatenate(
        [ei[0].reshape(NW, E_PER_T),
         jnp.zeros((NW, pad), jnp.int32)], axis=1).reshape(NW * NCHUNK, CHUNK)
    dst2d = jnp.concatenate(
        [ei[1].reshape(NW, E_PER_T),
         jnp.full((NW, pad), DUMMY_DST, jnp.int32)],
        axis=1).reshape(NW * NCHUNK, CHUNK)
    zeros_chunk = jnp.zeros((CHUNK, D), jnp.float32)
    agg2 = _sc_aggregate(x, src2d, dst2d, zeros_chunk)
    return _mlp(x, agg2, W1, b1, W2, b2, eps)
